# baseline (speedup 1.0000x reference)
import functools

import jax
import jax.numpy as jnp
from jax import lax
from jax.experimental import pallas as pl
from jax.experimental.pallas import tpu as pltpu
from jax.experimental.pallas import tpu_sc as plsc

B = 16
T = 576
N_ROWS = B * T
D_IN = 768
K = 8192
D_C = 256

TM = 1152
TN = 2048
M_TILES = N_ROWS // TM
N_TILES = K // TN

SC_CORES = 2
SC_SUBCORES = 16
SC_WORKERS = SC_CORES * SC_SUBCORES
ROWS_PER_W = N_ROWS // SC_WORKERS
IDX_CHUNK = 96
N_CHUNKS = ROWS_PER_W // IDX_CHUNK
CNT_W = 128
ROWS_PER_SUB = K // SC_SUBCORES
ZERO_ROWS = 64


def _cb_norm_body(cb_ref, cbnT_ref):
    cb = cb_ref[...]
    nrm = jnp.sqrt(jnp.sum(cb * cb, axis=1, keepdims=True))
    cbn = cb / jnp.maximum(nrm, 1e-12)
    cbnT_ref[...] = cbn.T


def _cb_norm(codebook_w):
    return pl.pallas_call(
        _cb_norm_body,
        grid=(N_TILES,),
        in_specs=[pl.BlockSpec((TN, D_C), lambda j: (j, 0))],
        out_specs=pl.BlockSpec((D_C, TN), lambda j: (0, j)),
        out_shape=jax.ShapeDtypeStruct((D_C, K), jnp.float32),
    )(codebook_w)


def _in_proj_body(z_ref, v_ref, g_ref, b_ref, enc_ref, encn_ref):
    zb = z_ref[0]
    v = v_ref[...]
    vss = jnp.sum(v * v, axis=1, keepdims=True)
    w = v * (g_ref[...] / jnp.sqrt(vss))
    zt = jnp.dot(w, zb, preferred_element_type=jnp.float32) + b_ref[...]
    e = zt.T
    nrm = jnp.sqrt(jnp.sum(e * e, axis=1, keepdims=True))
    enc_ref[...] = e
    encn_ref[...] = e / jnp.maximum(nrm, 1e-12)


def _in_proj(z, v_in2, g_in2, b_in2):
    return pl.pallas_call(
        _in_proj_body,
        grid=(B,),
        in_specs=[
            pl.BlockSpec((1, D_IN, T), lambda b: (b, 0, 0)),
            pl.BlockSpec((D_C, D_IN), lambda b: (0, 0)),
            pl.BlockSpec((D_C, 1), lambda b: (0, 0)),
            pl.BlockSpec((D_C, 1), lambda b: (0, 0)),
        ],
        out_specs=[
            pl.BlockSpec((T, D_C), lambda b: (b, 0)),
            pl.BlockSpec((T, D_C), lambda b: (b, 0)),
        ],
        out_shape=[
            jax.ShapeDtypeStruct((N_ROWS, D_C), jnp.float32),
            jax.ShapeDtypeStruct((N_ROWS, D_C), jnp.float32),
        ],
    )(z, v_in2, g_in2, b_in2)


def _dist_body(encn_ref, cbnT_ref, dist_ref, idx_ref,
               cbT_s, ess_ref, css_ref, minv_ref, mini_ref):
    m = pl.program_id(0)
    n = pl.program_id(1)

    @pl.when(n == 0)
    def _():
        enc0 = encn_ref[...]
        ess_ref[...] = jnp.sum(enc0 * enc0, axis=1, keepdims=True)

    @pl.when(m == 0)
    def _():
        cbT0 = cbnT_ref[...]
        cbT_s[:, pl.ds(n * TN, TN)] = cbT0
        css_ref[:, pl.ds(n * TN, TN)] = jnp.sum(cbT0 * cbT0, axis=0,
                                                keepdims=True)

    enc = encn_ref[...]
    cbT = cbT_s[:, pl.ds(n * TN, TN)]
    dots = jnp.dot(enc, cbT, preferred_element_type=jnp.float32)
    ess = ess_ref[...]
    css = css_ref[:, pl.ds(n * TN, TN)]
    dist = ess - 2.0 * dots + css
    dist_ref[...] = dist

    tmin = jnp.min(dist, axis=1, keepdims=True)
    lane = lax.broadcasted_iota(jnp.int32, dist.shape, 1)
    targ = jnp.min(jnp.where(dist == tmin, lane, jnp.int32(2 ** 30)),
                   axis=1, keepdims=True) + n * TN

    @pl.when(n == 0)
    def _():
        minv_ref[...] = tmin
        mini_ref[...] = targ

    @pl.when(n > 0)
    def _():
        better = tmin < minv_ref[...]
        minv_ref[...] = jnp.where(better, tmin, minv_ref[...])
        mini_ref[...] = jnp.where(better, targ, mini_ref[...])

    @pl.when(n == N_TILES - 1)
    def _():
        idx_ref[...] = jnp.broadcast_to(mini_ref[...], (TM, 128))


def _dist_argmin(enc_n, cb_nT):
    return pl.pallas_call(
        _dist_body,
        grid=(M_TILES, N_TILES),
        in_specs=[
            pl.BlockSpec((TM, D_C), lambda m, n: (m, 0)),
            pl.BlockSpec((D_C, TN),
                         lambda m, n: (0, jnp.where(m == 0, n, 0))),
        ],
        out_specs=[
            pl.BlockSpec((TM, TN), lambda m, n: (m, n)),
            pl.BlockSpec((TM, 128), lambda m, n: (m, 0)),
        ],
        out_shape=[
            jax.ShapeDtypeStruct((N_ROWS, K), jnp.float32),
            jax.ShapeDtypeStruct((N_ROWS, 128), jnp.int32),
        ],
        scratch_shapes=[
            pltpu.VMEM((D_C, K), jnp.float32),
            pltpu.VMEM((TM, 1), jnp.float32),
            pltpu.VMEM((1, K), jnp.float32),
            pltpu.VMEM((TM, 1), jnp.float32),
            pltpu.VMEM((TM, 1), jnp.int32),
        ],
    )(enc_n, cb_nT)


def _sc_body(cb_hbm, idx_hbm, counts_hbm, zq_hbm,
             i0, i1, i2, rows_v, ones_v, zero_v, shared, sem):
    c = lax.axis_index("c")
    s = lax.axis_index("s")
    wid = s * SC_CORES + c
    base = wid * ROWS_PER_W
    chunks = [i0, i1, i2]

    for j in range(N_CHUNKS):
        pltpu.sync_copy(idx_hbm.at[pl.ds(base + j * IDX_CHUNK, IDX_CHUNK)],
                        chunks[j])

    pltpu.async_copy(cb_hbm.at[chunks[0]], rows_v, sem)

    def _fill_ones(i, _):
        ones_v[i] = jnp.ones((CNT_W,), jnp.float32)
        return 0

    lax.fori_loop(0, IDX_CHUNK, _fill_ones, 0)

    def _fill_zero(i, _):
        zero_v[i] = jnp.zeros((CNT_W,), jnp.float32)
        return 0

    lax.fori_loop(0, ZERO_ROWS, _fill_zero, 0)

    for j in range(ROWS_PER_SUB // ZERO_ROWS):
        pltpu.sync_copy(zero_v, shared.at[pl.ds(s * ROWS_PER_SUB + j * ZERO_ROWS,
                                                ZERO_ROWS)])
    plsc.subcore_barrier()

    for j in range(N_CHUNKS):
        pltpu.sync_copy(ones_v, shared.at[chunks[j]], add=True)
    plsc.subcore_barrier()

    pltpu.sync_copy(shared.at[pl.ds(s * ROWS_PER_SUB, ROWS_PER_SUB)],
                    counts_hbm.at[c].at[pl.ds(s * ROWS_PER_SUB, ROWS_PER_SUB)])

    for j in range(N_CHUNKS):
        pltpu.make_async_copy(cb_hbm.at[chunks[j]], rows_v, sem).wait()
        pltpu.sync_copy(rows_v,
                        zq_hbm.at[pl.ds(base + j * IDX_CHUNK, IDX_CHUNK)])
        if j + 1 < N_CHUNKS:
            pltpu.async_copy(cb_hbm.at[chunks[j + 1]], rows_v, sem)


@functools.cache
def _sc_gather_counts_fn():
    return functools.partial(
        pl.kernel,
        mesh=plsc.VectorSubcoreMesh(core_axis_name="c", subcore_axis_name="s"),
        out_type=[
            jax.ShapeDtypeStruct((SC_CORES, K, CNT_W), jnp.float32),
            jax.ShapeDtypeStruct((N_ROWS, D_C), jnp.float32),
        ],
        scratch_types=[
            pltpu.VMEM((IDX_CHUNK,), jnp.int32),
            pltpu.VMEM((IDX_CHUNK,), jnp.int32),
            pltpu.VMEM((IDX_CHUNK,), jnp.int32),
            pltpu.VMEM((IDX_CHUNK, D_C), jnp.float32),
            pltpu.VMEM((IDX_CHUNK, CNT_W), jnp.float32),
            pltpu.VMEM((ZERO_ROWS, CNT_W), jnp.float32),
            pltpu.VMEM_SHARED((K, CNT_W), jnp.float32),
            pltpu.SemaphoreType.DMA,
        ],
    )(_sc_body)


def _out_proj_body(zq_ref, enc_ref, cnt_ref, v_ref, g_ref, b_ref,
                   zout_ref, stats_ref, acc_ref):
    bidx = pl.program_id(0)
    v = v_ref[...]
    vss = jnp.sum(v * v, axis=1, keepdims=True)
    w = v * (g_ref[...] / jnp.sqrt(vss))
    zq = zq_ref[...]
    zo = jnp.dot(w, zq.T, preferred_element_type=jnp.float32) + b_ref[...]
    zout_ref[...] = zo[None]

    d = enc_ref[...] - zq
    ssq = jnp.sum(d * d)

    @pl.when(bidx == 0)
    def _():
        acc_ref[0] = ssq

    @pl.when(bidx > 0)
    def _():
        acc_ref[0] = acc_ref[0] + ssq

    @pl.when(bidx == B - 1)
    def _():
        counts = cnt_ref[0] + cnt_ref[1]
        avg = counts / jnp.float32(N_ROWS)
        ent = -jnp.sum(avg * jnp.log(avg + 1e-10))
        perp = jnp.exp(ent)
        cluster = counts * (jnp.float32(1.0) - jnp.float32(0.99))
        active = jnp.sum((cluster > 2.0).astype(jnp.float32))
        vq_loss = acc_ref[0] * (1.25 / jnp.float32(B * D_C * T))
        col = lax.broadcasted_iota(jnp.int32, (1, 128), 1)
        stats = jnp.where(col == 0, vq_loss,
                          jnp.where(col == 1, perp,
                                    jnp.where(col == 2, active, 0.0)))
        stats_ref[...] = stats


def _out_proj(zq, enc, counts, v_out2, g_out2, b_out2):
    return pl.pallas_call(
        _out_proj_body,
        grid=(B,),
        in_specs=[
            pl.BlockSpec((T, D_C), lambda b: (b, 0)),
            pl.BlockSpec((T, D_C), lambda b: (b, 0)),
            pl.BlockSpec((SC_CORES, 8, K // 8), lambda b: (0, 0, 0)),
            pl.BlockSpec((D_IN, D_C), lambda b: (0, 0)),
            pl.BlockSpec((D_IN, 1), lambda b: (0, 0)),
            pl.BlockSpec((D_IN, 1), lambda b: (0, 0)),
        ],
        out_specs=[
            pl.BlockSpec((1, D_IN, T), lambda b: (b, 0, 0)),
            pl.BlockSpec((1, 128), lambda b: (0, 0)),
        ],
        out_shape=[
            jax.ShapeDtypeStruct((B, D_IN, T), jnp.float32),
            jax.ShapeDtypeStruct((1, 128), jnp.float32),
        ],
        scratch_shapes=[pltpu.SMEM((1,), jnp.float32)],
    )(zq, enc, counts, v_out2, g_out2, b_out2)


def kernel(z, codebook_w, v_in, g_in, b_in, v_out, g_out, b_out):
    v_in2 = v_in[:, :, 0]
    g_in2 = g_in.reshape(D_C, 1)
    b_in2 = b_in.reshape(D_C, 1)
    v_out2 = v_out[:, :, 0]
    g_out2 = g_out.reshape(D_IN, 1)
    b_out2 = b_out.reshape(D_IN, 1)

    enc, enc_n = _in_proj(z, v_in2, g_in2, b_in2)
    cb_nT = _cb_norm(codebook_w)
    dist, idx128 = _dist_argmin(enc_n, cb_nT)
    indices_flat = idx128[:, 0]
    counts, zq = _sc_gather_counts_fn()(codebook_w, indices_flat)
    counts8 = counts[:, :, 0].reshape(SC_CORES, 8, K // 8)
    z_out, stats = _out_proj(zq, enc, counts8, v_out2, g_out2, b_out2)

    indices = indices_flat.reshape(B, T)
    return (z_out, indices, dist, stats[0, 0], stats[0, 1], stats[0, 2])


# speedup vs baseline: 1.0130x; 1.0130x over previous
import functools

import jax
import jax.numpy as jnp
from jax import lax
from jax.experimental import pallas as pl
from jax.experimental.pallas import tpu as pltpu
from jax.experimental.pallas import tpu_sc as plsc

B = 16
T = 576
N_ROWS = B * T
D_IN = 768
K = 8192
D_C = 256

TM = 2304
TN = 2048
M_TILES = N_ROWS // TM
N_TILES = K // TN

SC_CORES = 2
SC_SUBCORES = 16
SC_WORKERS = SC_CORES * SC_SUBCORES
ROWS_PER_W = N_ROWS // SC_WORKERS
IDX_CHUNK = 96
N_CHUNKS = ROWS_PER_W // IDX_CHUNK
CNT_W = 128
ROWS_PER_SUB = K // SC_SUBCORES
ZERO_ROWS = 64


def _cb_norm_body(cb_ref, cbnT_ref):
    cb = cb_ref[...]
    nrm = jnp.sqrt(jnp.sum(cb * cb, axis=1, keepdims=True))
    cbn = cb / jnp.maximum(nrm, 1e-12)
    cbnT_ref[...] = cbn.T


def _cb_norm(codebook_w):
    return pl.pallas_call(
        _cb_norm_body,
        grid=(N_TILES,),
        in_specs=[pl.BlockSpec((TN, D_C), lambda j: (j, 0))],
        out_specs=pl.BlockSpec((D_C, TN), lambda j: (0, j)),
        out_shape=jax.ShapeDtypeStruct((D_C, K), jnp.float32),
    )(codebook_w)


def _in_proj_body(z_ref, v_ref, g_ref, b_ref, enc_ref, encn_ref):
    zb = z_ref[0]
    v = v_ref[...]
    vss = jnp.sum(v * v, axis=1, keepdims=True)
    w = v * (g_ref[...] / jnp.sqrt(vss))
    zt = jnp.dot(w, zb, preferred_element_type=jnp.float32) + b_ref[...]
    e = zt.T
    nrm = jnp.sqrt(jnp.sum(e * e, axis=1, keepdims=True))
    enc_ref[...] = e
    encn_ref[...] = e / jnp.maximum(nrm, 1e-12)


def _in_proj(z, v_in2, g_in2, b_in2):
    return pl.pallas_call(
        _in_proj_body,
        grid=(B,),
        in_specs=[
            pl.BlockSpec((1, D_IN, T), lambda b: (b, 0, 0)),
            pl.BlockSpec((D_C, D_IN), lambda b: (0, 0)),
            pl.BlockSpec((D_C, 1), lambda b: (0, 0)),
            pl.BlockSpec((D_C, 1), lambda b: (0, 0)),
        ],
        out_specs=[
            pl.BlockSpec((T, D_C), lambda b: (b, 0)),
            pl.BlockSpec((T, D_C), lambda b: (b, 0)),
        ],
        out_shape=[
            jax.ShapeDtypeStruct((N_ROWS, D_C), jnp.float32),
            jax.ShapeDtypeStruct((N_ROWS, D_C), jnp.float32),
        ],
    )(z, v_in2, g_in2, b_in2)


def _dist_body(encn_ref, cbnT_ref, dist_ref, idx_ref,
               ess_ref, css_ref, minv_ref, mini_ref):
    m = pl.program_id(0)
    n = pl.program_id(1)

    @pl.when(n == 0)
    def _():
        enc0 = encn_ref[...]
        ess_ref[...] = jnp.sum(enc0 * enc0, axis=1, keepdims=True)

    @pl.when(m == 0)
    def _():
        cbT0 = cbnT_ref[...]
        css_ref[:, pl.ds(n * TN, TN)] = jnp.sum(cbT0 * cbT0, axis=0,
                                                keepdims=True)

    enc = encn_ref[...]
    cbT = cbnT_ref[...]
    dots = jnp.dot(enc, cbT, preferred_element_type=jnp.float32)
    ess = ess_ref[...]
    css = css_ref[:, pl.ds(n * TN, TN)]
    dist = ess - 2.0 * dots + css
    dist_ref[...] = dist

    tmin = jnp.min(dist, axis=1, keepdims=True)
    lane = lax.broadcasted_iota(jnp.int32, dist.shape, 1)
    targ = jnp.min(jnp.where(dist == tmin, lane, jnp.int32(2 ** 30)),
                   axis=1, keepdims=True) + n * TN

    @pl.when(n == 0)
    def _():
        minv_ref[...] = tmin
        mini_ref[...] = targ

    @pl.when(n > 0)
    def _():
        better = tmin < minv_ref[...]
        minv_ref[...] = jnp.where(better, tmin, minv_ref[...])
        mini_ref[...] = jnp.where(better, targ, mini_ref[...])

    @pl.when(n == N_TILES - 1)
    def _():
        idx_ref[...] = jnp.broadcast_to(mini_ref[...], (TM, 128))


def _dist_argmin(enc_n, cb_nT):
    return pl.pallas_call(
        _dist_body,
        grid=(M_TILES, N_TILES),
        in_specs=[
            pl.BlockSpec((TM, D_C), lambda m, n: (m, 0)),
            pl.BlockSpec((D_C, TN), lambda m, n: (0, n)),
        ],
        out_specs=[
            pl.BlockSpec((TM, TN), lambda m, n: (m, n)),
            pl.BlockSpec((TM, 128), lambda m, n: (m, 0)),
        ],
        out_shape=[
            jax.ShapeDtypeStruct((N_ROWS, K), jnp.float32),
            jax.ShapeDtypeStruct((N_ROWS, 128), jnp.int32),
        ],
        scratch_shapes=[
            pltpu.VMEM((TM, 1), jnp.float32),
            pltpu.VMEM((1, K), jnp.float32),
            pltpu.VMEM((TM, 1), jnp.float32),
            pltpu.VMEM((TM, 1), jnp.int32),
        ],
    )(enc_n, cb_nT)


def _sc_body(cb_hbm, idx_hbm, counts_hbm, zq_hbm,
             i0, i1, i2, rows_v, ones_v, zero_v, shared, sem):
    c = lax.axis_index("c")
    s = lax.axis_index("s")
    wid = s * SC_CORES + c
    base = wid * ROWS_PER_W
    chunks = [i0, i1, i2]

    for j in range(N_CHUNKS):
        pltpu.sync_copy(idx_hbm.at[pl.ds(base + j * IDX_CHUNK, IDX_CHUNK)],
                        chunks[j])

    pltpu.async_copy(cb_hbm.at[chunks[0]], rows_v, sem)

    def _fill_ones(i, _):
        ones_v[i] = jnp.ones((CNT_W,), jnp.float32)
        return 0

    lax.fori_loop(0, IDX_CHUNK, _fill_ones, 0)

    def _fill_zero(i, _):
        zero_v[i] = jnp.zeros((CNT_W,), jnp.float32)
        return 0

    lax.fori_loop(0, ZERO_ROWS, _fill_zero, 0)

    for j in range(ROWS_PER_SUB // ZERO_ROWS):
        pltpu.sync_copy(zero_v, shared.at[pl.ds(s * ROWS_PER_SUB + j * ZERO_ROWS,
                                                ZERO_ROWS)])
    plsc.subcore_barrier()

    for j in range(N_CHUNKS):
        pltpu.sync_copy(ones_v, shared.at[chunks[j]], add=True)
    plsc.subcore_barrier()

    pltpu.sync_copy(shared.at[pl.ds(s * ROWS_PER_SUB, ROWS_PER_SUB)],
                    counts_hbm.at[c].at[pl.ds(s * ROWS_PER_SUB, ROWS_PER_SUB)])

    for j in range(N_CHUNKS):
        pltpu.make_async_copy(cb_hbm.at[chunks[j]], rows_v, sem).wait()
        pltpu.sync_copy(rows_v,
                        zq_hbm.at[pl.ds(base + j * IDX_CHUNK, IDX_CHUNK)])
        if j + 1 < N_CHUNKS:
            pltpu.async_copy(cb_hbm.at[chunks[j + 1]], rows_v, sem)


@functools.cache
def _sc_gather_counts_fn():
    return functools.partial(
        pl.kernel,
        mesh=plsc.VectorSubcoreMesh(core_axis_name="c", subcore_axis_name="s"),
        out_type=[
            jax.ShapeDtypeStruct((SC_CORES, K, CNT_W), jnp.float32),
            jax.ShapeDtypeStruct((N_ROWS, D_C), jnp.float32),
        ],
        scratch_types=[
            pltpu.VMEM((IDX_CHUNK,), jnp.int32),
            pltpu.VMEM((IDX_CHUNK,), jnp.int32),
            pltpu.VMEM((IDX_CHUNK,), jnp.int32),
            pltpu.VMEM((IDX_CHUNK, D_C), jnp.float32),
            pltpu.VMEM((IDX_CHUNK, CNT_W), jnp.float32),
            pltpu.VMEM((ZERO_ROWS, CNT_W), jnp.float32),
            pltpu.VMEM_SHARED((K, CNT_W), jnp.float32),
            pltpu.SemaphoreType.DMA,
        ],
    )(_sc_body)


def _out_proj_body(zq_ref, enc_ref, cnt_ref, v_ref, g_ref, b_ref,
                   zout_ref, stats_ref, acc_ref):
    bidx = pl.program_id(0)
    v = v_ref[...]
    vss = jnp.sum(v * v, axis=1, keepdims=True)
    w = v * (g_ref[...] / jnp.sqrt(vss))
    zq = zq_ref[...]
    zo = jnp.dot(w, zq.T, preferred_element_type=jnp.float32) + b_ref[...]
    zout_ref[...] = zo[None]

    d = enc_ref[...] - zq
    ssq = jnp.sum(d * d)

    @pl.when(bidx == 0)
    def _():
        acc_ref[0] = ssq

    @pl.when(bidx > 0)
    def _():
        acc_ref[0] = acc_ref[0] + ssq

    @pl.when(bidx == B - 1)
    def _():
        counts = cnt_ref[0] + cnt_ref[1]
        avg = counts / jnp.float32(N_ROWS)
        ent = -jnp.sum(avg * jnp.log(avg + 1e-10))
        perp = jnp.exp(ent)
        cluster = counts * (jnp.float32(1.0) - jnp.float32(0.99))
        active = jnp.sum((cluster > 2.0).astype(jnp.float32))
        vq_loss = acc_ref[0] * (1.25 / jnp.float32(B * D_C * T))
        col = lax.broadcasted_iota(jnp.int32, (1, 128), 1)
        stats = jnp.where(col == 0, vq_loss,
                          jnp.where(col == 1, perp,
                                    jnp.where(col == 2, active, 0.0)))
        stats_ref[...] = stats


def _out_proj(zq, enc, counts, v_out2, g_out2, b_out2):
    return pl.pallas_call(
        _out_proj_body,
        grid=(B,),
        in_specs=[
            pl.BlockSpec((T, D_C), lambda b: (b, 0)),
            pl.BlockSpec((T, D_C), lambda b: (b, 0)),
            pl.BlockSpec((SC_CORES, 8, K // 8), lambda b: (0, 0, 0)),
            pl.BlockSpec((D_IN, D_C), lambda b: (0, 0)),
            pl.BlockSpec((D_IN, 1), lambda b: (0, 0)),
            pl.BlockSpec((D_IN, 1), lambda b: (0, 0)),
        ],
        out_specs=[
            pl.BlockSpec((1, D_IN, T), lambda b: (b, 0, 0)),
            pl.BlockSpec((1, 128), lambda b: (0, 0)),
        ],
        out_shape=[
            jax.ShapeDtypeStruct((B, D_IN, T), jnp.float32),
            jax.ShapeDtypeStruct((1, 128), jnp.float32),
        ],
        scratch_shapes=[pltpu.SMEM((1,), jnp.float32)],
    )(zq, enc, counts, v_out2, g_out2, b_out2)


def kernel(z, codebook_w, v_in, g_in, b_in, v_out, g_out, b_out):
    v_in2 = v_in[:, :, 0]
    g_in2 = g_in.reshape(D_C, 1)
    b_in2 = b_in.reshape(D_C, 1)
    v_out2 = v_out[:, :, 0]
    g_out2 = g_out.reshape(D_IN, 1)
    b_out2 = b_out.reshape(D_IN, 1)

    enc, enc_n = _in_proj(z, v_in2, g_in2, b_in2)
    cb_nT = _cb_norm(codebook_w)
    dist, idx128 = _dist_argmin(enc_n, cb_nT)
    indices_flat = idx128[:, 0]
    counts, zq = _sc_gather_counts_fn()(codebook_w, indices_flat)
    counts8 = counts[:, :, 0].reshape(SC_CORES, 8, K // 8)
    z_out, stats = _out_proj(zq, enc, counts8, v_out2, g_out2, b_out2)

    indices = indices_flat.reshape(B, T)
    return (z_out, indices, dist, stats[0, 0], stats[0, 1], stats[0, 2])


# speedup vs baseline: 1.0297x; 1.0165x over previous
import functools

import jax
import jax.numpy as jnp
from jax import lax
from jax.experimental import pallas as pl
from jax.experimental.pallas import tpu as pltpu
from jax.experimental.pallas import tpu_sc as plsc

B = 16
T = 576
N_ROWS = B * T
D_IN = 768
K = 8192
D_C = 256

TM = 2304
TN = 2048
M_TILES = N_ROWS // TM
N_TILES = K // TN

SC_CORES = 2
SC_SUBCORES = 16
SC_WORKERS = SC_CORES * SC_SUBCORES
ROWS_PER_W = N_ROWS // SC_WORKERS
IDX_CHUNK = 96
N_CHUNKS = ROWS_PER_W // IDX_CHUNK
CNT_W = 128
ROWS_PER_SUB = K // SC_SUBCORES
ZERO_ROWS = 64


CBT = K // B


def _in_proj_body(z_ref, v_ref, g_ref, b_ref, cb_ref,
                  enc_ref, encn_ref, cbnT_ref):
    zb = z_ref[0]
    v = v_ref[...]
    vss = jnp.sum(v * v, axis=1, keepdims=True)
    w = v * (g_ref[...] / jnp.sqrt(vss))
    zt = jnp.dot(w, zb, preferred_element_type=jnp.float32) + b_ref[...]
    e = zt.T
    nrm = jnp.sqrt(jnp.sum(e * e, axis=1, keepdims=True))
    enc_ref[...] = e
    encn_ref[...] = e / jnp.maximum(nrm, 1e-12)

    cb = cb_ref[...]
    cnrm = jnp.sqrt(jnp.sum(cb * cb, axis=1, keepdims=True))
    cbn = cb / jnp.maximum(cnrm, 1e-12)
    cbnT_ref[...] = cbn.T


def _in_proj(z, v_in2, g_in2, b_in2, codebook_w):
    return pl.pallas_call(
        _in_proj_body,
        grid=(B,),
        in_specs=[
            pl.BlockSpec((1, D_IN, T), lambda b: (b, 0, 0)),
            pl.BlockSpec((D_C, D_IN), lambda b: (0, 0)),
            pl.BlockSpec((D_C, 1), lambda b: (0, 0)),
            pl.BlockSpec((D_C, 1), lambda b: (0, 0)),
            pl.BlockSpec((CBT, D_C), lambda b: (b, 0)),
        ],
        out_specs=[
            pl.BlockSpec((T, D_C), lambda b: (b, 0)),
            pl.BlockSpec((T, D_C), lambda b: (b, 0)),
            pl.BlockSpec((D_C, CBT), lambda b: (0, b)),
        ],
        out_shape=[
            jax.ShapeDtypeStruct((N_ROWS, D_C), jnp.float32),
            jax.ShapeDtypeStruct((N_ROWS, D_C), jnp.float32),
            jax.ShapeDtypeStruct((D_C, K), jnp.float32),
        ],
    )(z, v_in2, g_in2, b_in2, codebook_w)


def _dist_body(encn_ref, cbnT_ref, dist_ref, idx_ref,
               ess_ref, css_ref, minv_ref, mini_ref):
    m = pl.program_id(0)
    n = pl.program_id(1)

    @pl.when(n == 0)
    def _():
        enc0 = encn_ref[...]
        ess_ref[...] = jnp.sum(enc0 * enc0, axis=1, keepdims=True)

    @pl.when(m == 0)
    def _():
        cbT0 = cbnT_ref[...]
        css_ref[:, pl.ds(n * TN, TN)] = jnp.sum(cbT0 * cbT0, axis=0,
                                                keepdims=True)

    enc = encn_ref[...]
    cbT = cbnT_ref[...]
    dots = jnp.dot(enc, cbT, preferred_element_type=jnp.float32)
    ess = ess_ref[...]
    css = css_ref[:, pl.ds(n * TN, TN)]
    dist = ess - 2.0 * dots + css
    dist_ref[...] = dist

    tmin = jnp.min(dist, axis=1, keepdims=True)
    lane = lax.broadcasted_iota(jnp.int32, dist.shape, 1)
    targ = jnp.min(jnp.where(dist == tmin, lane, jnp.int32(2 ** 30)),
                   axis=1, keepdims=True) + n * TN

    @pl.when(n == 0)
    def _():
        minv_ref[...] = tmin
        mini_ref[...] = targ

    @pl.when(n > 0)
    def _():
        better = tmin < minv_ref[...]
        minv_ref[...] = jnp.where(better, tmin, minv_ref[...])
        mini_ref[...] = jnp.where(better, targ, mini_ref[...])

    @pl.when(n == N_TILES - 1)
    def _():
        idx_ref[...] = jnp.broadcast_to(mini_ref[...], (TM, 128))


def _dist_argmin(enc_n, cb_nT):
    return pl.pallas_call(
        _dist_body,
        grid=(M_TILES, N_TILES),
        in_specs=[
            pl.BlockSpec((TM, D_C), lambda m, n: (m, 0)),
            pl.BlockSpec((D_C, TN), lambda m, n: (0, n)),
        ],
        out_specs=[
            pl.BlockSpec((TM, TN), lambda m, n: (m, n)),
            pl.BlockSpec((TM, 128), lambda m, n: (m, 0)),
        ],
        out_shape=[
            jax.ShapeDtypeStruct((N_ROWS, K), jnp.float32),
            jax.ShapeDtypeStruct((N_ROWS, 128), jnp.int32),
        ],
        scratch_shapes=[
            pltpu.VMEM((TM, 1), jnp.float32),
            pltpu.VMEM((1, K), jnp.float32),
            pltpu.VMEM((TM, 1), jnp.float32),
            pltpu.VMEM((TM, 1), jnp.int32),
        ],
    )(enc_n, cb_nT)


def _sc_body(cb_hbm, idx_hbm, counts_hbm, zq_hbm,
             i0, i1, i2, rows_v, ones_v, zero_v, shared, sem):
    c = lax.axis_index("c")
    s = lax.axis_index("s")
    wid = s * SC_CORES + c
    base = wid * ROWS_PER_W
    chunks = [i0, i1, i2]

    for j in range(N_CHUNKS):
        pltpu.sync_copy(idx_hbm.at[pl.ds(base + j * IDX_CHUNK, IDX_CHUNK)],
                        chunks[j])

    pltpu.async_copy(cb_hbm.at[chunks[0]], rows_v, sem)

    def _fill_ones(i, _):
        ones_v[i] = jnp.ones((CNT_W,), jnp.float32)
        return 0

    lax.fori_loop(0, IDX_CHUNK, _fill_ones, 0)

    def _fill_zero(i, _):
        zero_v[i] = jnp.zeros((CNT_W,), jnp.float32)
        return 0

    lax.fori_loop(0, ZERO_ROWS, _fill_zero, 0)

    for j in range(ROWS_PER_SUB // ZERO_ROWS):
        pltpu.sync_copy(zero_v, shared.at[pl.ds(s * ROWS_PER_SUB + j * ZERO_ROWS,
                                                ZERO_ROWS)])
    plsc.subcore_barrier()

    for j in range(N_CHUNKS):
        pltpu.sync_copy(ones_v, shared.at[chunks[j]], add=True)
    plsc.subcore_barrier()

    pltpu.sync_copy(shared.at[pl.ds(s * ROWS_PER_SUB, ROWS_PER_SUB)],
                    counts_hbm.at[c].at[pl.ds(s * ROWS_PER_SUB, ROWS_PER_SUB)])

    for j in range(N_CHUNKS):
        pltpu.make_async_copy(cb_hbm.at[chunks[j]], rows_v, sem).wait()
        pltpu.sync_copy(rows_v,
                        zq_hbm.at[pl.ds(base + j * IDX_CHUNK, IDX_CHUNK)])
        if j + 1 < N_CHUNKS:
            pltpu.async_copy(cb_hbm.at[chunks[j + 1]], rows_v, sem)


@functools.cache
def _sc_gather_counts_fn():
    return functools.partial(
        pl.kernel,
        mesh=plsc.VectorSubcoreMesh(core_axis_name="c", subcore_axis_name="s"),
        out_type=[
            jax.ShapeDtypeStruct((SC_CORES, K, CNT_W), jnp.float32),
            jax.ShapeDtypeStruct((N_ROWS, D_C), jnp.float32),
        ],
        scratch_types=[
            pltpu.VMEM((IDX_CHUNK,), jnp.int32),
            pltpu.VMEM((IDX_CHUNK,), jnp.int32),
            pltpu.VMEM((IDX_CHUNK,), jnp.int32),
            pltpu.VMEM((IDX_CHUNK, D_C), jnp.float32),
            pltpu.VMEM((IDX_CHUNK, CNT_W), jnp.float32),
            pltpu.VMEM((ZERO_ROWS, CNT_W), jnp.float32),
            pltpu.VMEM_SHARED((K, CNT_W), jnp.float32),
            pltpu.SemaphoreType.DMA,
        ],
    )(_sc_body)


def _out_proj_body(zq_ref, enc_ref, cnt_ref, v_ref, g_ref, b_ref,
                   zout_ref, stats_ref, acc_ref):
    bidx = pl.program_id(0)
    v = v_ref[...]
    vss = jnp.sum(v * v, axis=1, keepdims=True)
    w = v * (g_ref[...] / jnp.sqrt(vss))
    zq = zq_ref[...]
    zo = jnp.dot(w, zq.T, preferred_element_type=jnp.float32) + b_ref[...]
    zout_ref[...] = zo[None]

    d = enc_ref[...] - zq
    ssq = jnp.sum(d * d)

    @pl.when(bidx == 0)
    def _():
        acc_ref[0] = ssq

    @pl.when(bidx > 0)
    def _():
        acc_ref[0] = acc_ref[0] + ssq

    @pl.when(bidx == B - 1)
    def _():
        counts = cnt_ref[0] + cnt_ref[1]
        avg = counts / jnp.float32(N_ROWS)
        ent = -jnp.sum(avg * jnp.log(avg + 1e-10))
        perp = jnp.exp(ent)
        cluster = counts * (jnp.float32(1.0) - jnp.float32(0.99))
        active = jnp.sum((cluster > 2.0).astype(jnp.float32))
        vq_loss = acc_ref[0] * (1.25 / jnp.float32(B * D_C * T))
        col = lax.broadcasted_iota(jnp.int32, (1, 128), 1)
        stats = jnp.where(col == 0, vq_loss,
                          jnp.where(col == 1, perp,
                                    jnp.where(col == 2, active, 0.0)))
        stats_ref[...] = stats


def _out_proj(zq, enc, counts, v_out2, g_out2, b_out2):
    return pl.pallas_call(
        _out_proj_body,
        grid=(B,),
        in_specs=[
            pl.BlockSpec((T, D_C), lambda b: (b, 0)),
            pl.BlockSpec((T, D_C), lambda b: (b, 0)),
            pl.BlockSpec((SC_CORES, 8, K // 8), lambda b: (0, 0, 0)),
            pl.BlockSpec((D_IN, D_C), lambda b: (0, 0)),
            pl.BlockSpec((D_IN, 1), lambda b: (0, 0)),
            pl.BlockSpec((D_IN, 1), lambda b: (0, 0)),
        ],
        out_specs=[
            pl.BlockSpec((1, D_IN, T), lambda b: (b, 0, 0)),
            pl.BlockSpec((1, 128), lambda b: (0, 0)),
        ],
        out_shape=[
            jax.ShapeDtypeStruct((B, D_IN, T), jnp.float32),
            jax.ShapeDtypeStruct((1, 128), jnp.float32),
        ],
        scratch_shapes=[pltpu.SMEM((1,), jnp.float32)],
    )(zq, enc, counts, v_out2, g_out2, b_out2)


def kernel(z, codebook_w, v_in, g_in, b_in, v_out, g_out, b_out):
    v_in2 = v_in[:, :, 0]
    g_in2 = g_in.reshape(D_C, 1)
    b_in2 = b_in.reshape(D_C, 1)
    v_out2 = v_out[:, :, 0]
    g_out2 = g_out.reshape(D_IN, 1)
    b_out2 = b_out.reshape(D_IN, 1)

    enc, enc_n, cb_nT = _in_proj(z, v_in2, g_in2, b_in2, codebook_w)
    dist, idx128 = _dist_argmin(enc_n, cb_nT)
    indices_flat = idx128[:, 0]
    counts, zq = _sc_gather_counts_fn()(codebook_w, indices_flat)
    counts8 = counts[:, :, 0].reshape(SC_CORES, 8, K // 8)
    z_out, stats = _out_proj(zq, enc, counts8, v_out2, g_out2, b_out2)

    indices = indices_flat.reshape(B, T)
    return (z_out, indices, dist, stats[0, 0], stats[0, 1], stats[0, 2])
